# baseline (speedup 1.0000x reference)
import functools

import jax
import jax.numpy as jnp
from jax import lax
from jax.experimental import pallas as pl
from jax.experimental.pallas import tpu as pltpu
from jax.experimental.pallas import tpu_sc as plsc

_B = 8
_T = 4
_P = 1025
_H = 1280
_PH = _P * _H
_PR = 2
_NW = 32
_NCID = _P // _PR
_IDPW = _NCID // _NW
_NITEMS = _IDPW * _B
_NBUF = 4
_PC = 256
_NPC = _H // _PC
_L = 16


def _sc_body(hs, ids16, g16, emb1, tile, out,
             ids_v, g_v, emb_c, hs_buf, out_buf,
             stage, hs_tail, emb_t, out_tail, in_sem, out_sem):
    w = lax.axis_index("s") * 2 + lax.axis_index("c")

    pltpu.sync_copy(ids16, ids_v)
    pltpu.sync_copy(g16, g_v)
    gvec = g_v[...]
    omg = 1.0 - gvec
    idvec = ids_v[...]
    lanes = lax.iota(jnp.int32, 16)
    gnz = jnp.max(jnp.abs(gvec)) != 0.0

    def extract(b):
        return jnp.max(jnp.where(lanes == b, idvec, 0))

    for kk in range(_IDPW):
        pltpu.sync_copy(emb1.at[pl.ds((w + kk * _NW) * _PR * _H, _PR * _H)],
                        emb_c.at[kk])

    for kk in range(_IDPW):
        @plsc.parallel_loop(0, _PR * _H, step=_L, unroll=8)
        def _(i):
            emb_c[kk, pl.ds(i, _L)] = omg * emb_c[kk, pl.ds(i, _L)]

    def coords(it):
        kk = it // _B
        b = it % _B
        return kk, b, w + kk * _NW

    def issue(it, s):
        _, b, j = coords(it)
        pltpu.make_async_copy(hs.at[b, pl.ds(j * _PR, _PR), :, :],
                              hs_buf.at[s], in_sem.at[s]).start()

    def wait_in(s):
        pltpu.make_async_copy(hs.at[0, pl.ds(0, _PR), :, :],
                              hs_buf.at[s], in_sem.at[s]).wait()

    def compute(it, s):
        kk, b, j = coords(it)
        for r in range(_PR):
            @plsc.parallel_loop(0, _H, step=_L, unroll=8)
            def _(i):
                sl = pl.ds(i, _L)
                ev = emb_c[kk, pl.ds(r * _H + i, _L)]
                for t in range(_T):
                    out_buf[s, r, t, sl] = hs_buf[s, r, t, sl] + ev

        @pl.when(gnz)
        def _():
            row = extract(b)
            rbase = (row // 8) * 8
            rsel = row % 8
            for t in range(_T):
                for r in range(_PR):
                    for p in range(_NPC):
                        pltpu.sync_copy(
                            tile.at[pl.ds(rbase, 8),
                                    pl.ds(t * _PH + (j * _PR + r) * _H
                                          + p * _PC, _PC)],
                            stage)

                        @plsc.parallel_loop(0, _PC, step=_L, unroll=8)
                        def _(i):
                            out_buf[s, r, t, pl.ds(p * _PC + i, _L)] = (
                                out_buf[s, r, t, pl.ds(p * _PC + i, _L)]
                                + gvec * stage[rsel, pl.ds(i, _L)])

    def start_out(it, s):
        _, b, j = coords(it)
        pltpu.make_async_copy(out_buf.at[s], out.at[b, pl.ds(j * _PR, _PR), :, :],
                              out_sem.at[s]).start()

    def wait_out(s):
        pltpu.make_async_copy(out_buf.at[s], out.at[0, pl.ds(0, _PR), :, :],
                              out_sem.at[s]).wait()

    for s in range(_NBUF):
        issue(s, s)

    def group(gr, _):
        for s in range(_NBUF):
            it = gr * _NBUF + s
            wait_in(s)

            @pl.when(gr > 0)
            def _():
                wait_out(s)

            compute(it, s)
            start_out(it, s)

            @pl.when(it + _NBUF < _NITEMS)
            def _():
                issue(it + _NBUF, s)
        return 0

    lax.fori_loop(0, _NITEMS // _NBUF, group, 0)
    for s in range(_NBUF):
        wait_out(s)

    @pl.when(w < _B)
    def _():
        b_w = w
        for p in range(_NPC):
            pltpu.sync_copy(hs.at[b_w, pl.ds(_P - 1, 1), :,
                                  pl.ds(p * _PC, _PC)], hs_tail)
            pltpu.sync_copy(emb1.at[pl.ds((_P - 1) * _H + p * _PC, _PC)],
                            emb_t)

            @plsc.parallel_loop(0, _PC, step=_L, unroll=8)
            def _(i):
                sl = pl.ds(i, _L)
                ev = omg * emb_t[sl]
                for t in range(_T):
                    out_tail[0, t, sl] = hs_tail[0, t, sl] + ev

            @pl.when(gnz)
            def _():
                row = extract(b_w)
                rbase = (row // 8) * 8
                rsel = row % 8
                for t in range(_T):
                    pltpu.sync_copy(
                        tile.at[pl.ds(rbase, 8),
                                pl.ds(t * _PH + (_P - 1) * _H + p * _PC, _PC)],
                        stage)

                    @plsc.parallel_loop(0, _PC, step=_L, unroll=8)
                    def _(i):
                        sl = pl.ds(i, _L)
                        out_tail[0, t, sl] = (out_tail[0, t, sl]
                                              + gvec * stage[rsel, sl])

            pltpu.sync_copy(out_tail, out.at[b_w, pl.ds(_P - 1, 1), :,
                                             pl.ds(p * _PC, _PC)])


def kernel(hidden_state, aspect_ratio_ids, gate, embedding, tile_embedding):
    ids16 = jnp.zeros((16,), jnp.int32).at[:_B].set(
        aspect_ratio_ids.astype(jnp.int32))
    g16 = jnp.full((16,), jnp.tanh(gate[0]), jnp.float32)
    hs_t = jnp.transpose(hidden_state, (0, 2, 1, 3))
    emb1 = embedding.reshape(-1)

    mesh = plsc.VectorSubcoreMesh(core_axis_name="c", subcore_axis_name="s")
    f = functools.partial(
        pl.kernel,
        out_type=jax.ShapeDtypeStruct((_B, _P, _T, _H), hidden_state.dtype),
        mesh=mesh,
        scratch_types=[
            pltpu.VMEM((16,), jnp.int32),
            pltpu.VMEM((16,), jnp.float32),
            pltpu.VMEM((_IDPW, _PR * _H), jnp.float32),
            pltpu.VMEM((_NBUF, _PR, _T, _H), jnp.float32),
            pltpu.VMEM((_NBUF, _PR, _T, _H), jnp.float32),
            pltpu.VMEM((8, _PC), jnp.float32),
            pltpu.VMEM((1, _T, _PC), jnp.float32),
            pltpu.VMEM((_PC,), jnp.float32),
            pltpu.VMEM((1, _T, _PC), jnp.float32),
            pltpu.SemaphoreType.DMA((_NBUF,)),
            pltpu.SemaphoreType.DMA((_NBUF,)),
        ],
        compiler_params=pltpu.CompilerParams(use_tc_tiling_on_sc=True,
                                             needs_layout_passes=False),
    )(_sc_body)
    out_t = f(hs_t, ids16, g16, emb1, tile_embedding)
    return jnp.transpose(out_t, (0, 2, 1, 3))


# speedup vs baseline: 1.0533x; 1.0533x over previous
import functools

import jax
import jax.numpy as jnp
from jax import lax
from jax.experimental import pallas as pl
from jax.experimental.pallas import tpu as pltpu
from jax.experimental.pallas import tpu_sc as plsc

_B = 8
_T = 4
_P = 1025
_H = 1280
_PH = _P * _H
_PR = 2
_NW = 32
_NCID = _P // _PR
_IDPW = _NCID // _NW
_NITEMS = _IDPW * _B
_NBUF = 4
_L = 16


def _sc_body(hs, ids16, g16, emb1, tile, out,
             ids_v, g_v, hs_buf, emb_buf, out_buf,
             stage, hs_tail, emb_t, out_tail, in_sem, out_sem):
    w = lax.axis_index("s") * 2 + lax.axis_index("c")

    pltpu.sync_copy(ids16, ids_v)
    pltpu.sync_copy(g16, g_v)
    gvec = g_v[...]
    omg = 1.0 - gvec
    idvec = ids_v[...]
    lanes = lax.iota(jnp.int32, 16)
    gnz = jnp.max(jnp.abs(gvec)) != 0.0

    def extract(b):
        return jnp.max(jnp.where(lanes == b, idvec, 0))

    def coords(it):
        kk = it // _B
        b = it % _B
        return kk, b, w + kk * _NW

    def issue(it, s):
        _, b, j = coords(it)
        pltpu.make_async_copy(hs.at[b, pl.ds(j * _PR, _PR), :, :],
                              hs_buf.at[s], in_sem.at[s]).start()
        pltpu.make_async_copy(emb1.at[pl.ds(j * _PR * _H, _PR * _H)],
                              emb_buf.at[s], in_sem.at[s]).start()

    def wait_in(s):
        pltpu.make_async_copy(hs.at[0, pl.ds(0, _PR), :, :],
                              hs_buf.at[s], in_sem.at[s]).wait()
        pltpu.make_async_copy(emb1.at[pl.ds(0, _PR * _H)],
                              emb_buf.at[s], in_sem.at[s]).wait()

    def compute(it, s):
        _, b, j = coords(it)
        for r in range(_PR):
            @plsc.parallel_loop(0, _H, step=_L, unroll=8)
            def _(i):
                sl = pl.ds(i, _L)
                ev = omg * emb_buf[s, pl.ds(r * _H + i, _L)]
                for t in range(_T):
                    out_buf[s, r, t, sl] = hs_buf[s, r, t, sl] + ev

        @pl.when(gnz)
        def _():
            row = extract(b)
            rbase = (row // 8) * 8
            rsel = row % 8
            for t in range(_T):
                for r in range(_PR):
                    pltpu.sync_copy(
                        tile.at[pl.ds(rbase, 8),
                                pl.ds(t * _PH + (j * _PR + r) * _H, _H)],
                        stage)

                    @plsc.parallel_loop(0, _H, step=_L, unroll=8)
                    def _(i):
                        sl = pl.ds(i, _L)
                        out_buf[s, r, t, sl] = (out_buf[s, r, t, sl]
                                                + gvec * stage[rsel, sl])

    def start_out(it, s):
        _, b, j = coords(it)
        pltpu.make_async_copy(out_buf.at[s], out.at[b, pl.ds(j * _PR, _PR), :, :],
                              out_sem.at[s]).start()

    def wait_out(s):
        pltpu.make_async_copy(out_buf.at[s], out.at[0, pl.ds(0, _PR), :, :],
                              out_sem.at[s]).wait()

    for s in range(_NBUF):
        issue(s, s)

    def group(gr, _):
        for s in range(_NBUF):
            it = gr * _NBUF + s
            wait_in(s)

            @pl.when(gr > 0)
            def _():
                wait_out(s)

            compute(it, s)
            start_out(it, s)

            @pl.when(it + _NBUF < _NITEMS)
            def _():
                issue(it + _NBUF, s)
        return 0

    lax.fori_loop(0, _NITEMS // _NBUF, group, 0)
    for s in range(_NBUF):
        wait_out(s)

    @pl.when(w < _B)
    def _():
        b_w = w
        pltpu.sync_copy(hs.at[b_w, pl.ds(_P - 1, 1), :, :], hs_tail)
        pltpu.sync_copy(emb1.at[pl.ds((_P - 1) * _H, _H)], emb_t)

        @plsc.parallel_loop(0, _H, step=_L, unroll=8)
        def _(i):
            sl = pl.ds(i, _L)
            ev = omg * emb_t[sl]
            for t in range(_T):
                out_tail[0, t, sl] = hs_tail[0, t, sl] + ev

        @pl.when(gnz)
        def _():
            row = extract(b_w)
            rbase = (row // 8) * 8
            rsel = row % 8
            for t in range(_T):
                pltpu.sync_copy(
                    tile.at[pl.ds(rbase, 8),
                            pl.ds(t * _PH + (_P - 1) * _H, _H)], stage)

                @plsc.parallel_loop(0, _H, step=_L, unroll=8)
                def _(i):
                    sl = pl.ds(i, _L)
                    out_tail[0, t, sl] = (out_tail[0, t, sl]
                                          + gvec * stage[rsel, sl])

        pltpu.sync_copy(out_tail, out.at[b_w, pl.ds(_P - 1, 1), :, :])


def kernel(hidden_state, aspect_ratio_ids, gate, embedding, tile_embedding):
    ids16 = jnp.zeros((16,), jnp.int32).at[:_B].set(
        aspect_ratio_ids.astype(jnp.int32))
    g16 = jnp.full((16,), jnp.tanh(gate[0]), jnp.float32)
    hs_t = jnp.transpose(hidden_state, (0, 2, 1, 3))
    emb1 = embedding.reshape(-1)

    mesh = plsc.VectorSubcoreMesh(core_axis_name="c", subcore_axis_name="s")
    f = functools.partial(
        pl.kernel,
        out_type=jax.ShapeDtypeStruct((_B, _P, _T, _H), hidden_state.dtype),
        mesh=mesh,
        scratch_types=[
            pltpu.VMEM((16,), jnp.int32),
            pltpu.VMEM((16,), jnp.float32),
            pltpu.VMEM((_NBUF, _PR, _T, _H), jnp.float32),
            pltpu.VMEM((_NBUF, _PR * _H), jnp.float32),
            pltpu.VMEM((_NBUF, _PR, _T, _H), jnp.float32),
            pltpu.VMEM((8, _H), jnp.float32),
            pltpu.VMEM((1, _T, _H), jnp.float32),
            pltpu.VMEM((_H,), jnp.float32),
            pltpu.VMEM((1, _T, _H), jnp.float32),
            pltpu.SemaphoreType.DMA((_NBUF,)),
            pltpu.SemaphoreType.DMA((_NBUF,)),
        ],
        compiler_params=pltpu.CompilerParams(use_tc_tiling_on_sc=True,
                                             needs_layout_passes=False),
    )(_sc_body)
    out_t = f(hs_t, ids16, g16, emb1, tile_embedding)
    return jnp.transpose(out_t, (0, 2, 1, 3))
